# baseline (speedup 1.0000x reference)
import jax
import jax.numpy as jnp
from jax import lax
from jax.experimental import pallas as pl
from jax.experimental.pallas import tpu as pltpu
from jax.experimental.pallas import tpu_sc as plsc

N = 10000
E = 320000
D = 128
H = 8
DH = 16
EDGE_DIM = 16
D_HID = 4 * D

NC = 2
NS = 16
NW = NC * NS
L = 16
R = 320
NPAD = NW * R
CAP = 12288
CHUNK = 48
NCH = CAP // CHUNK
PCH = 96
NPCH = CAP // PCH
BLK = 2000
RD = R + 8

_f32 = jnp.float32
_i32 = jnp.int32


def _make_bin_kernel():
    mesh = plsc.VectorSubcoreMesh(core_axis_name="c", subcore_axis_name="s",
                                  num_cores=NC, num_subcores=NS)
    out_type = (
        jax.ShapeDtypeStruct((NW * CAP,), _i32),
        jax.ShapeDtypeStruct((NW * CAP,), _i32),
        jax.ShapeDtypeStruct((NW * CAP,), _i32),
        jax.ShapeDtypeStruct((NW * 8,), _i32),
    )
    scratch = [
        pltpu.VMEM((BLK,), _i32),
        pltpu.VMEM((BLK,), _i32),
        pltpu.VMEM((BLK,), _i32),
        pltpu.VMEM((BLK,), _i32),
        pltpu.VMEM((CAP + 2 * L,), _i32),
        pltpu.VMEM((CAP + 2 * L,), _i32),
        pltpu.VMEM((CAP + 2 * L,), _i32),
        pltpu.VMEM((L,), _i32),
    ] + [pltpu.SemaphoreType.DMA] * 4

    def body(src_hbm, dst_hbm, bsrc, bdstl, beid, bcnt,
             sb0, sb1, db0, db1, lsrc, ldstl, leid, cbuf,
             bs0, bs1, bd0, bd1):
        w = lax.axis_index("s") * NC + lax.axis_index("c")
        lo = w * R
        iota = lax.iota(_i32, L)
        zi = jnp.zeros((L,), _i32)
        pads = jnp.full((L,), R, _i32)

        def prefill(i, _):
            lsrc[pl.ds(i * L, L)] = zi
            ldstl[pl.ds(i * L, L)] = pads
            leid[pl.ds(i * L, L)] = zi
            return 0

        lax.fori_loop(0, (CAP + 2 * L) // L, prefill, 0)

        sbb = (sb0, sb1)
        dbb = (db0, db1)
        bss = (bs0, bs1)
        bds = (bd0, bd1)

        def _bissue(b, t):
            pltpu.async_copy(src_hbm.at[pl.ds(b * BLK, BLK)], sbb[t], bss[t])
            pltpu.async_copy(dst_hbm.at[pl.ds(b * BLK, BLK)], dbb[t], bds[t])

        def _bwait(t):
            pltpu.make_async_copy(
                src_hbm.at[pl.ds(0, BLK)], sbb[t], bss[t]).wait()
            pltpu.make_async_copy(
                dst_hbm.at[pl.ds(0, BLK)], dbb[t], bds[t]).wait()

        _bissue(0, 0)
        _bissue(1, 1)

        def bstep(b, t, cntv):
            sbuf, dbuf = sbb[t], dbb[t]
            _bwait(t)

            def vec(i, cntv):
                dl = dbuf[pl.ds(i * L, L)] - lo
                s = sbuf[pl.ds(i * L, L)]
                m = (dl >= 0) & (dl < R)
                mi = jnp.where(m, 1, 0)
                pos = cntv + plsc.cumsum(mi) - 1
                idx = jnp.where(m, pos, CAP + L)
                plsc.store_scatter(ldstl, [idx], dl)
                plsc.store_scatter(lsrc, [idx], s)
                plsc.store_scatter(leid, [idx], b * BLK + i * L + iota)
                cntv = cntv + plsc.all_reduce_population_count(m)
                return jnp.minimum(cntv, CAP)

            cntv = lax.fori_loop(0, BLK // L, vec, cntv)

            @pl.when(b + 2 < E // BLK)
            def _():
                _bissue(b + 2, t)

            return cntv

        def bpair(j, cntv):
            cntv = bstep(2 * j, 0, cntv)
            return bstep(2 * j + 1, 1, cntv)

        cntv = lax.fori_loop(0, E // BLK // 2, bpair, jnp.zeros((L,), _i32))
        cbuf[pl.ds(0, L)] = cntv
        pltpu.sync_copy(cbuf.at[pl.ds(0, 8)], bcnt.at[pl.ds(w * 8, 8)])
        pltpu.sync_copy(lsrc.at[pl.ds(0, CAP)], bsrc.at[pl.ds(w * CAP, CAP)])
        pltpu.sync_copy(ldstl.at[pl.ds(0, CAP)], bdstl.at[pl.ds(w * CAP, CAP)])
        pltpu.sync_copy(leid.at[pl.ds(0, CAP)], beid.at[pl.ds(w * CAP, CAP)])

    return pl.kernel(
        body, out_type=out_type, mesh=mesh, scratch_types=scratch,
        compiler_params=pltpu.CompilerParams(needs_layout_passes=False))


def _make_attn_kernel():
    mesh = plsc.VectorSubcoreMesh(core_axis_name="c", subcore_axis_name="s",
                                  num_cores=NC, num_subcores=NS)
    out_type = jax.ShapeDtypeStruct((NPAD * D,), _f32)
    scratch = [
        pltpu.VMEM(((R + 1) * D,), _f32),
        pltpu.VMEM(((R + 1) * D,), _f32),
        pltpu.VMEM((CHUNK, 2 * D), _f32),
        pltpu.VMEM((CHUNK, 2 * D), _f32),
        pltpu.VMEM((CHUNK, D), _f32),
        pltpu.VMEM((CHUNK, D), _f32),
        pltpu.VMEM((CHUNK,), _i32),
        pltpu.VMEM((CHUNK,), _i32),
        pltpu.VMEM((CHUNK,), _i32),
        pltpu.VMEM((CHUNK,), _i32),
        pltpu.VMEM((CHUNK,), _i32),
        pltpu.VMEM((CHUNK,), _i32),
        pltpu.VMEM((H * RD + L,), _f32),
        pltpu.VMEM((L,), _i32),
        pltpu.VMEM((L,), _f32),
    ] + [pltpu.SemaphoreType.DMA] * 10

    def body(qpf, kvp, eemb, bsrc, bdstl, beid, bcnt, aggf,
             qa, ag, kv0, kv1, eb0, eb1, sb0, sb1, ib0, ib1, db0, db1, den, cbuf, tmp,
             smkv0, smkv1, sme0, sme1, sms0, sms1, smi0, smi1, smd0, smd1):
        w = lax.axis_index("s") * NC + lax.axis_index("c")
        lo = w * R
        iota = lax.iota(_i32, L)
        zf = jnp.zeros((L,), _f32)

        kvb = (kv0, kv1)
        eb = (eb0, eb1)
        sb = (sb0, sb1)
        ib = (ib0, ib1)
        db = (db0, db1)
        smkv = (smkv0, smkv1)
        sme = (sme0, sme1)
        sms = (sms0, sms1)
        smi = (smi0, smi1)
        smd = (smd0, smd1)

        pltpu.sync_copy(qpf.at[pl.ds(lo * D, R * D)], qa.at[pl.ds(0, R * D)])
        pltpu.sync_copy(bcnt.at[pl.ds(w * 8, 8)], cbuf.at[pl.ds(0, 8)])
        cnt = cbuf[pl.ds(0, L)][0]
        npair = (cnt + 2 * CHUNK - 1) // (2 * CHUNK)
        nch_w = npair * 2

        def zden(i, _):
            den[pl.ds(i * L, L)] = zf
            return 0

        lax.fori_loop(0, (H * RD + L) // L, zden, 0)

        def zagg(i, _):
            ag[pl.ds(i * L, L)] = zf
            return 0

        lax.fori_loop(0, (R + 1) * D // L, zagg, 0)

        def _issue_idx(c, s):
            pltpu.async_copy(bsrc.at[pl.ds(w * CAP + c * CHUNK, CHUNK)],
                             sb[s], sms[s])
            pltpu.async_copy(beid.at[pl.ds(w * CAP + c * CHUNK, CHUNK)],
                             ib[s], smi[s])

        def _wait_idx(s):
            pltpu.make_async_copy(
                bsrc.at[pl.ds(0, CHUNK)], sb[s], sms[s]).wait()
            pltpu.make_async_copy(
                beid.at[pl.ds(0, CHUNK)], ib[s], smi[s]).wait()

        def _issue_dl(c, s):
            pltpu.async_copy(bdstl.at[pl.ds(w * CAP + c * CHUNK, CHUNK)],
                             db[s], smd[s])

        def _wait_dl(s):
            pltpu.make_async_copy(
                bdstl.at[pl.ds(0, CHUNK)], db[s], smd[s]).wait()

        def _issue_g(s):
            pltpu.async_copy(kvp.at[sb[s]], kvb[s], smkv[s])
            pltpu.async_copy(eemb.at[ib[s]], eb[s], sme[s])

        def _wait_g(s):
            pltpu.make_async_copy(
                kvp.at[sb[s]], kvb[s], smkv[s]).wait()
            pltpu.make_async_copy(
                eemb.at[ib[s]], eb[s], sme[s]).wait()

        stride8 = iota * 8
        perm_adj = iota ^ 1
        perm_pack = (iota & 7) * 2
        perm_head = iota >> 1
        hsel = iota < 8

        def _compute(s):
            kvbuf, ebuf, dlbuf = kvb[s], eb[s], db[s]

            def grp(g, _):
                dstl_vec = dlbuf[pl.ds(g * L, L)]
                for e in range(L):
                    dstl = dstl_vec[e]
                    abase = dstl * D
                    row = jnp.full((L,), g * L + e, _i32)
                    acc = zf
                    for dd in range(8):
                        av = stride8 + dd
                        qv = plsc.load_gather(qa, [abase + av])
                        kv = plsc.load_gather(kvbuf, [row, av])
                        ev = plsc.load_gather(ebuf, [row, av])
                        acc = acc + qv * (kv + ev)
                    tmp[pl.ds(0, L)] = acc
                    acc2 = acc + plsc.load_gather(tmp, [perm_adj])
                    tmp[pl.ds(0, L)] = acc2
                    packed = plsc.load_gather(tmp, [perm_pack])
                    exv = jnp.exp(packed * 0.25)
                    didx = jnp.where(hsel, iota * RD + dstl, H * RD)
                    plsc.addupdate_scatter(den, [didx], exv)
                    tmp[pl.ds(0, L)] = exv
                    alpha = plsc.load_gather(tmp, [perm_head])
                    for dd in range(8):
                        av = stride8 + dd
                        vv = plsc.load_gather(kvbuf, [row, D + av])
                        ev = plsc.load_gather(ebuf, [row, av])
                        plsc.addupdate_scatter(
                            ag, [abase + av], alpha * (vv + ev))
                return 0

            lax.fori_loop(0, CHUNK // L, grp, 0)

        def step(c, s):
            _wait_g(s)

            @pl.when(c + 2 < nch_w)
            def _():
                _issue_idx(c + 2, s)

            _wait_dl(s)
            _compute(s)

            @pl.when(c + 2 < nch_w)
            def _():
                _issue_dl(c + 2, s)
                _wait_idx(s)
                _issue_g(s)

        _issue_idx(0, 0)
        _issue_dl(0, 0)
        _issue_idx(1, 1)
        _issue_dl(1, 1)
        _wait_idx(0)
        _issue_g(0)
        _wait_idx(1)
        _issue_g(1)

        def pair(j, _):
            step(2 * j, 0)
            step(2 * j + 1, 1)
            return 0

        lax.fori_loop(0, npair, pair, 0)

        def norm(t, _):
            rv = t * L + iota
            for h in range(H):
                rd = plsc.load_gather(den, [h * RD + rv])
                rcp = 1.0 / (rd + 1e-16)
                for dd in range(DH):
                    dvec = h * DH + ((dd + iota) & (DH - 1))
                    av = plsc.load_gather(ag, [rv * D + dvec])
                    plsc.store_scatter(ag, [rv * D + dvec], av * rcp)
            return 0

        lax.fori_loop(0, R // L, norm, 0)
        pltpu.sync_copy(ag.at[pl.ds(0, R * D)], aggf.at[pl.ds(lo * D, R * D)])

    return pl.kernel(
        body, out_type=out_type, mesh=mesh, scratch_types=scratch,
        compiler_params=pltpu.CompilerParams(needs_layout_passes=False))


_NB = 256
_EB = 1280


def _proj_nodes_body(x, wq, wk, wv, bq, bk, bv, q, kv):
    xv = x[...]
    q[...] = jnp.dot(xv, wq[...], preferred_element_type=_f32) + bq[...]
    kv[:, :D] = jnp.dot(xv, wk[...], preferred_element_type=_f32) + bk[...]
    kv[:, D:] = jnp.dot(xv, wv[...], preferred_element_type=_f32) + bv[...]


def _proj_nodes(f, wq, wk, wv, bq, bk, bv):
    full = lambda s: pl.BlockSpec(s, lambda i: (0, 0))
    return pl.pallas_call(
        _proj_nodes_body,
        grid=(NPAD // _NB,),
        in_specs=[pl.BlockSpec((_NB, D), lambda i: (i, 0)),
                  full((D, D)), full((D, D)), full((D, D)),
                  full((1, D)), full((1, D)), full((1, D))],
        out_specs=[pl.BlockSpec((_NB, D), lambda i: (i, 0)),
                   pl.BlockSpec((_NB, 2 * D), lambda i: (i, 0))],
        out_shape=[jax.ShapeDtypeStruct((NPAD, D), _f32),
                   jax.ShapeDtypeStruct((NPAD, 2 * D), _f32)],
    )(f, wq, wk, wv, bq, bk, bv)


def _proj_edges_body(x, we, be, o):
    o[...] = jnp.dot(x[...], we[...], preferred_element_type=_f32) + be[...]


def _proj_edges(edge_attr, we, be):
    return pl.pallas_call(
        _proj_edges_body,
        grid=(E // _EB,),
        in_specs=[pl.BlockSpec((_EB, EDGE_DIM), lambda i: (i, 0)),
                  pl.BlockSpec((EDGE_DIM, D), lambda i: (0, 0)),
                  pl.BlockSpec((1, D), lambda i: (0, 0))],
        out_specs=pl.BlockSpec((_EB, D), lambda i: (i, 0)),
        out_shape=jax.ShapeDtypeStruct((E, D), _f32),
    )(edge_attr, we, be)


def _ln(x, g, b):
    mu = jnp.mean(x, axis=-1, keepdims=True)
    var = jnp.mean((x - mu) ** 2, axis=-1, keepdims=True)
    return (x - mu) / jnp.sqrt(var + 1e-5) * g + b


def _post_body(f, agg, wo, bo, g1, b1, w1, b1m, w2, b2m, g2, b2, o):
    att = jnp.dot(agg[...], wo[...], preferred_element_type=_f32) + bo[...]
    x = _ln(f[...] + att, g1[...], b1[...])
    hmid = jnp.maximum(jnp.dot(x, w1[...], preferred_element_type=_f32) + b1m[...], 0.0)
    hh = jnp.dot(hmid, w2[...], preferred_element_type=_f32) + b2m[...]
    o[...] = _ln(x + hh, g2[...], b2[...])


def _post(f, agg, wo, bo, g1, b1, w1, b1m, w2, b2m, g2, b2):
    full = lambda s: pl.BlockSpec(s, lambda i: (0, 0))
    return pl.pallas_call(
        _post_body,
        grid=(NPAD // _NB,),
        in_specs=[pl.BlockSpec((_NB, D), lambda i: (i, 0)),
                  pl.BlockSpec((_NB, D), lambda i: (i, 0)),
                  full((D, D)), full((1, D)), full((1, D)), full((1, D)),
                  full((D, D_HID)), full((1, D_HID)),
                  full((D_HID, D)), full((1, D)),
                  full((1, D)), full((1, D))],
        out_specs=pl.BlockSpec((_NB, D), lambda i: (i, 0)),
        out_shape=jax.ShapeDtypeStruct((NPAD, D), _f32),
    )(f, agg, wo, bo, g1, b1, w1, b1m, w2, b2m, g2, b2)


_bin_kernel = _make_bin_kernel()
_attn_kernel = _make_attn_kernel()


def kernel(feats, edge_index, edge_attr, params):
    src = edge_index[0]
    dst = edge_index[1]
    bsrc, bdstl, beid, bcnt = _bin_kernel(src, dst)
    f = jnp.pad(feats, ((0, NPAD - N), (0, 0)))
    outs = []
    for p in params:
        r2 = lambda a: a.reshape(1, -1)
        q, kv = _proj_nodes(f, p['Wq'], p['Wk'], p['Wv'],
                            r2(p['bq']), r2(p['bk']), r2(p['bv']))
        eeb = _proj_edges(edge_attr, p['We'], r2(p['be']))
        aggf = _attn_kernel(q.reshape(-1), kv, eeb, bsrc, bdstl, beid, bcnt)
        f = _post(f, aggf.reshape(NPAD, D), p['Wo'], r2(p['bo']),
                  r2(p['g1']), r2(p['b1']), p['W1'], r2(p['b1m']),
                  p['W2'], r2(p['b2m']), r2(p['g2']), r2(p['b2']))
        outs.append(f[:N])
    return jnp.stack(outs, axis=0), edge_index, edge_attr


# speedup vs baseline: 1.9518x; 1.9518x over previous
import jax
import jax.numpy as jnp
from jax import lax
from jax.experimental import pallas as pl
from jax.experimental.pallas import tpu as pltpu
from jax.experimental.pallas import tpu_sc as plsc

N = 10000
E = 320000
D = 128
H = 8
DH = 16
EDGE_DIM = 16
D_HID = 4 * D

NC = 2
NS = 16
NW = NC * NS
L = 16
R = 320
NPAD = NW * R
CAP = 12288
CHUNK = 32
NCH = CAP // CHUNK
PCH = 96
NPCH = CAP // PCH
BLK = 2000
RD = R + 8

_f32 = jnp.float32
_i32 = jnp.int32


def _make_bin_kernel():
    mesh = plsc.VectorSubcoreMesh(core_axis_name="c", subcore_axis_name="s",
                                  num_cores=NC, num_subcores=NS)
    out_type = (
        jax.ShapeDtypeStruct((NW * CAP,), _i32),
        jax.ShapeDtypeStruct((NW * CAP,), _i32),
        jax.ShapeDtypeStruct((NW * CAP,), _i32),
        jax.ShapeDtypeStruct((NW * 8,), _i32),
    )
    scratch = [
        pltpu.VMEM((BLK,), _i32),
        pltpu.VMEM((BLK,), _i32),
        pltpu.VMEM((BLK,), _i32),
        pltpu.VMEM((BLK,), _i32),
        pltpu.VMEM((CAP + 2 * L,), _i32),
        pltpu.VMEM((CAP + 2 * L,), _i32),
        pltpu.VMEM((CAP + 2 * L,), _i32),
        pltpu.VMEM((L,), _i32),
    ] + [pltpu.SemaphoreType.DMA] * 4

    def body(src_hbm, dst_hbm, bsrc, bdstl, beid, bcnt,
             sb0, sb1, db0, db1, lsrc, ldstl, leid, cbuf,
             bs0, bs1, bd0, bd1):
        w = lax.axis_index("s") * NC + lax.axis_index("c")
        lo = w * R
        iota = lax.iota(_i32, L)
        zi = jnp.zeros((L,), _i32)
        pads = jnp.full((L,), R, _i32)

        def prefill(i, _):
            lsrc[pl.ds(i * L, L)] = zi
            ldstl[pl.ds(i * L, L)] = pads
            leid[pl.ds(i * L, L)] = zi
            return 0

        lax.fori_loop(0, (CAP + 2 * L) // L, prefill, 0)

        sbb = (sb0, sb1)
        dbb = (db0, db1)
        bss = (bs0, bs1)
        bds = (bd0, bd1)

        def _bissue(b, t):
            pltpu.async_copy(src_hbm.at[pl.ds(b * BLK, BLK)], sbb[t], bss[t])
            pltpu.async_copy(dst_hbm.at[pl.ds(b * BLK, BLK)], dbb[t], bds[t])

        def _bwait(t):
            pltpu.make_async_copy(
                src_hbm.at[pl.ds(0, BLK)], sbb[t], bss[t]).wait()
            pltpu.make_async_copy(
                dst_hbm.at[pl.ds(0, BLK)], dbb[t], bds[t]).wait()

        _bissue(0, 0)
        _bissue(1, 1)

        def bstep(b, t, cntv):
            sbuf, dbuf = sbb[t], dbb[t]
            _bwait(t)

            def vec(i, cntv):
                dl = dbuf[pl.ds(i * L, L)] - lo
                s = sbuf[pl.ds(i * L, L)]
                m = (dl >= 0) & (dl < R)
                mi = jnp.where(m, 1, 0)
                pos = cntv + plsc.cumsum(mi) - 1
                idx = jnp.where(m, pos, CAP + L)
                plsc.store_scatter(ldstl, [idx], dl)
                plsc.store_scatter(lsrc, [idx], s)
                plsc.store_scatter(leid, [idx], b * BLK + i * L + iota)
                cntv = cntv + plsc.all_reduce_population_count(m)
                return jnp.minimum(cntv, CAP)

            cntv = lax.fori_loop(0, BLK // L, vec, cntv)

            @pl.when(b + 2 < E // BLK)
            def _():
                _bissue(b + 2, t)

            return cntv

        def bpair(j, cntv):
            cntv = bstep(2 * j, 0, cntv)
            return bstep(2 * j + 1, 1, cntv)

        cntv = lax.fori_loop(0, E // BLK // 2, bpair, jnp.zeros((L,), _i32))
        cbuf[pl.ds(0, L)] = cntv
        pltpu.sync_copy(cbuf.at[pl.ds(0, 8)], bcnt.at[pl.ds(w * 8, 8)])
        pltpu.sync_copy(lsrc.at[pl.ds(0, CAP)], bsrc.at[pl.ds(w * CAP, CAP)])
        pltpu.sync_copy(ldstl.at[pl.ds(0, CAP)], bdstl.at[pl.ds(w * CAP, CAP)])
        pltpu.sync_copy(leid.at[pl.ds(0, CAP)], beid.at[pl.ds(w * CAP, CAP)])

    return pl.kernel(
        body, out_type=out_type, mesh=mesh, scratch_types=scratch,
        compiler_params=pltpu.CompilerParams(needs_layout_passes=False))


def _make_attn_kernel():
    mesh = plsc.VectorSubcoreMesh(core_axis_name="c", subcore_axis_name="s",
                                  num_cores=NC, num_subcores=NS)
    out_type = jax.ShapeDtypeStruct((NPAD * D,), _f32)
    scratch = [
        pltpu.VMEM(((R + 1) * D,), _f32),
        pltpu.VMEM(((R + 1) * D,), _f32),
        pltpu.VMEM((CHUNK, 2 * D), _f32),
        pltpu.VMEM((CHUNK, 2 * D), _f32),
        pltpu.VMEM((CHUNK, D), _f32),
        pltpu.VMEM((CHUNK, D), _f32),
        pltpu.VMEM((CHUNK,), _i32),
        pltpu.VMEM((CHUNK,), _i32),
        pltpu.VMEM((CHUNK,), _i32),
        pltpu.VMEM((CHUNK,), _i32),
        pltpu.VMEM((CHUNK,), _i32),
        pltpu.VMEM((CHUNK,), _i32),
        pltpu.VMEM((H * RD,), _f32),
        pltpu.VMEM((L,), _i32),
    ] + [pltpu.SemaphoreType.DMA] * 10

    def body(qpf, kvp, eemb, bsrc, bdstl, beid, bcnt, aggf,
             qa, ag, kv0, kv1, eb0, eb1, sb0, sb1, ib0, ib1, db0, db1, den, cbuf,
             smkv0, smkv1, sme0, sme1, sms0, sms1, smi0, smi1, smd0, smd1):
        w = lax.axis_index("s") * NC + lax.axis_index("c")
        lo = w * R
        iota = lax.iota(_i32, L)
        zf = jnp.zeros((L,), _f32)

        kvb = (kv0, kv1)
        eb = (eb0, eb1)
        sb = (sb0, sb1)
        ib = (ib0, ib1)
        db = (db0, db1)
        smkv = (smkv0, smkv1)
        sme = (sme0, sme1)
        sms = (sms0, sms1)
        smi = (smi0, smi1)
        smd = (smd0, smd1)

        pltpu.sync_copy(qpf.at[pl.ds(lo * D, R * D)], qa.at[pl.ds(0, R * D)])
        pltpu.sync_copy(bcnt.at[pl.ds(w * 8, 8)], cbuf.at[pl.ds(0, 8)])
        cnt = cbuf[pl.ds(0, L)][0]
        npair = (cnt + 2 * CHUNK - 1) // (2 * CHUNK)
        nch_w = npair * 2

        def zden(i, _):
            den[pl.ds(i * L, L)] = zf
            return 0

        lax.fori_loop(0, H * RD // L, zden, 0)

        def zagg(i, _):
            ag[pl.ds(i * L, L)] = zf
            return 0

        lax.fori_loop(0, (R + 1) * D // L, zagg, 0)

        def _issue_idx(c, s):
            pltpu.async_copy(bsrc.at[pl.ds(w * CAP + c * CHUNK, CHUNK)],
                             sb[s], sms[s])
            pltpu.async_copy(beid.at[pl.ds(w * CAP + c * CHUNK, CHUNK)],
                             ib[s], smi[s])

        def _wait_idx(s):
            pltpu.make_async_copy(
                bsrc.at[pl.ds(0, CHUNK)], sb[s], sms[s]).wait()
            pltpu.make_async_copy(
                beid.at[pl.ds(0, CHUNK)], ib[s], smi[s]).wait()

        def _issue_dl(c, s):
            pltpu.async_copy(bdstl.at[pl.ds(w * CAP + c * CHUNK, CHUNK)],
                             db[s], smd[s])

        def _wait_dl(s):
            pltpu.make_async_copy(
                bdstl.at[pl.ds(0, CHUNK)], db[s], smd[s]).wait()

        def _issue_g(s):
            pltpu.async_copy(kvp.at[sb[s]], kvb[s], smkv[s])
            pltpu.async_copy(eemb.at[ib[s]], eb[s], sme[s])

        def _wait_g(s):
            pltpu.make_async_copy(
                kvp.at[sb[s]], kvb[s], smkv[s]).wait()
            pltpu.make_async_copy(
                eemb.at[ib[s]], eb[s], sme[s]).wait()

        def _compute(s):
            kvbuf, ebuf, dlbuf = kvb[s], eb[s], db[s]

            def grp(g, _):
                dstl = dlbuf[pl.ds(g * L, L)]
                rowv = g * L + iota
                abase = dstl * D

                def hloop(h, _):
                    hb = h * DH
                    acc = zf
                    for dd in range(DH):
                        dvec = hb + ((dd + iota) & (DH - 1))
                        qv = plsc.load_gather(qa, [abase + dvec])
                        kv = plsc.load_gather(kvbuf, [rowv, dvec])
                        ev = plsc.load_gather(ebuf, [rowv, dvec])
                        acc = acc + qv * (kv + ev)
                    exv = jnp.exp(acc * 0.25)
                    plsc.addupdate_scatter(den, [h * RD + dstl], exv)
                    for dd in range(DH):
                        dvec = hb + ((dd + iota) & (DH - 1))
                        vv = plsc.load_gather(kvbuf, [rowv, D + dvec])
                        ev = plsc.load_gather(ebuf, [rowv, dvec])
                        plsc.addupdate_scatter(
                            ag, [abase + dvec], exv * (vv + ev))
                    return 0

                lax.fori_loop(0, H, hloop, 0)
                return 0

            lax.fori_loop(0, CHUNK // L, grp, 0)

        def step(c, s):
            _wait_g(s)

            @pl.when(c + 2 < nch_w)
            def _():
                _issue_idx(c + 2, s)

            _wait_dl(s)
            _compute(s)

            @pl.when(c + 2 < nch_w)
            def _():
                _issue_dl(c + 2, s)
                _wait_idx(s)
                _issue_g(s)

        _issue_idx(0, 0)
        _issue_dl(0, 0)
        _issue_idx(1, 1)
        _issue_dl(1, 1)
        _wait_idx(0)
        _issue_g(0)
        _wait_idx(1)
        _issue_g(1)

        def pair(j, _):
            step(2 * j, 0)
            step(2 * j + 1, 1)
            return 0

        lax.fori_loop(0, npair, pair, 0)

        def norm(t, _):
            rv = t * L + iota
            for h in range(H):
                rd = plsc.load_gather(den, [h * RD + rv])
                rcp = 1.0 / (rd + 1e-16)
                for dd in range(DH):
                    dvec = h * DH + ((dd + iota) & (DH - 1))
                    av = plsc.load_gather(ag, [rv * D + dvec])
                    plsc.store_scatter(ag, [rv * D + dvec], av * rcp)
            return 0

        lax.fori_loop(0, R // L, norm, 0)
        pltpu.sync_copy(ag.at[pl.ds(0, R * D)], aggf.at[pl.ds(lo * D, R * D)])

    return pl.kernel(
        body, out_type=out_type, mesh=mesh, scratch_types=scratch,
        compiler_params=pltpu.CompilerParams(needs_layout_passes=False))


_NB = 256
_EB = 1280


def _proj_nodes_body(x, wq, wk, wv, bq, bk, bv, q, kv):
    xv = x[...]
    q[...] = jnp.dot(xv, wq[...], preferred_element_type=_f32) + bq[...]
    kv[:, :D] = jnp.dot(xv, wk[...], preferred_element_type=_f32) + bk[...]
    kv[:, D:] = jnp.dot(xv, wv[...], preferred_element_type=_f32) + bv[...]


def _proj_nodes(f, wq, wk, wv, bq, bk, bv):
    full = lambda s: pl.BlockSpec(s, lambda i: (0, 0))
    return pl.pallas_call(
        _proj_nodes_body,
        grid=(NPAD // _NB,),
        in_specs=[pl.BlockSpec((_NB, D), lambda i: (i, 0)),
                  full((D, D)), full((D, D)), full((D, D)),
                  full((1, D)), full((1, D)), full((1, D))],
        out_specs=[pl.BlockSpec((_NB, D), lambda i: (i, 0)),
                   pl.BlockSpec((_NB, 2 * D), lambda i: (i, 0))],
        out_shape=[jax.ShapeDtypeStruct((NPAD, D), _f32),
                   jax.ShapeDtypeStruct((NPAD, 2 * D), _f32)],
    )(f, wq, wk, wv, bq, bk, bv)


def _proj_edges_body(x, we, be, o):
    o[...] = jnp.dot(x[...], we[...], preferred_element_type=_f32) + be[...]


def _proj_edges(edge_attr, we, be):
    return pl.pallas_call(
        _proj_edges_body,
        grid=(E // _EB,),
        in_specs=[pl.BlockSpec((_EB, EDGE_DIM), lambda i: (i, 0)),
                  pl.BlockSpec((EDGE_DIM, D), lambda i: (0, 0)),
                  pl.BlockSpec((1, D), lambda i: (0, 0))],
        out_specs=pl.BlockSpec((_EB, D), lambda i: (i, 0)),
        out_shape=jax.ShapeDtypeStruct((E, D), _f32),
    )(edge_attr, we, be)


def _ln(x, g, b):
    mu = jnp.mean(x, axis=-1, keepdims=True)
    var = jnp.mean((x - mu) ** 2, axis=-1, keepdims=True)
    return (x - mu) / jnp.sqrt(var + 1e-5) * g + b


def _post_body(f, agg, wo, bo, g1, b1, w1, b1m, w2, b2m, g2, b2, o):
    att = jnp.dot(agg[...], wo[...], preferred_element_type=_f32) + bo[...]
    x = _ln(f[...] + att, g1[...], b1[...])
    hmid = jnp.maximum(jnp.dot(x, w1[...], preferred_element_type=_f32) + b1m[...], 0.0)
    hh = jnp.dot(hmid, w2[...], preferred_element_type=_f32) + b2m[...]
    o[...] = _ln(x + hh, g2[...], b2[...])


def _post(f, agg, wo, bo, g1, b1, w1, b1m, w2, b2m, g2, b2):
    full = lambda s: pl.BlockSpec(s, lambda i: (0, 0))
    return pl.pallas_call(
        _post_body,
        grid=(NPAD // _NB,),
        in_specs=[pl.BlockSpec((_NB, D), lambda i: (i, 0)),
                  pl.BlockSpec((_NB, D), lambda i: (i, 0)),
                  full((D, D)), full((1, D)), full((1, D)), full((1, D)),
                  full((D, D_HID)), full((1, D_HID)),
                  full((D_HID, D)), full((1, D)),
                  full((1, D)), full((1, D))],
        out_specs=pl.BlockSpec((_NB, D), lambda i: (i, 0)),
        out_shape=jax.ShapeDtypeStruct((NPAD, D), _f32),
    )(f, agg, wo, bo, g1, b1, w1, b1m, w2, b2m, g2, b2)


_bin_kernel = _make_bin_kernel()
_attn_kernel = _make_attn_kernel()


def kernel(feats, edge_index, edge_attr, params):
    src = edge_index[0]
    dst = edge_index[1]
    bsrc, bdstl, beid, bcnt = _bin_kernel(src, dst)
    f = jnp.pad(feats, ((0, NPAD - N), (0, 0)))
    outs = []
    for p in params:
        r2 = lambda a: a.reshape(1, -1)
        q, kv = _proj_nodes(f, p['Wq'], p['Wk'], p['Wv'],
                            r2(p['bq']), r2(p['bk']), r2(p['bv']))
        eeb = _proj_edges(edge_attr, p['We'], r2(p['be']))
        aggf = _attn_kernel(q.reshape(-1), kv, eeb, bsrc, bdstl, beid, bcnt)
        f = _post(f, aggf.reshape(NPAD, D), p['Wo'], r2(p['bo']),
                  r2(p['g1']), r2(p['b1']), p['W1'], r2(p['b1m']),
                  p['W2'], r2(p['b2m']), r2(p['g2']), r2(p['b2']))
        outs.append(f[:N])
    return jnp.stack(outs, axis=0), edge_index, edge_attr
